# baseline (speedup 1.0000x reference)
import jax
import jax.numpy as jnp
from jax import lax
from jax.experimental import pallas as pl
from jax.experimental.pallas import tpu as pltpu
from jax.experimental.pallas import tpu_sc as plsc

N_NODES = 10000
N_EDGES = 320000
FEAT = 128
NUM_CORES = 2
NUM_SUBCORES = 16
NUM_TILES = NUM_CORES * NUM_SUBCORES
GROUP = 80
N_PAD = 10240
EDGES_PER_TILE = N_PAD
E_PAD = NUM_TILES * EDGES_PER_TILE
GROUPS = EDGES_PER_TILE // GROUP
ROWS_PER_TILE = N_PAD // NUM_SUBCORES
ZROWS = 16


def _sc_body(src_hbm, dst_hbm, state_hbm, out0, out1,
             idx_s, idx_d, rows, zbuf, sem_i, sem_g, sem_s, shared):
    cid = lax.axis_index("c")
    sid = lax.axis_index("s")
    wid = cid * NUM_SUBCORES + sid

    zeros16 = jnp.zeros((16,), jnp.float32)

    def _zrow(r, _):
        def _zcol(j, _):
            zbuf[r, pl.ds(j * 16, 16)] = zeros16
            return 0
        return lax.fori_loop(0, FEAT // 16, _zcol, 0)

    lax.fori_loop(0, ZROWS, _zrow, 0)

    row0 = sid * ROWS_PER_TILE
    for b in range(ROWS_PER_TILE // ZROWS):
        pltpu.sync_copy(zbuf, shared.at[pl.ds(row0 + b * ZROWS, ZROWS)])
    plsc.subcore_barrier()

    ebase = wid * EDGES_PER_TILE

    def _fire_idx(g, slot):
        off = ebase + g * GROUP
        pltpu.async_copy(src_hbm.at[pl.ds(off, GROUP)], idx_s.at[slot],
                         sem_i.at[slot])
        pltpu.async_copy(dst_hbm.at[pl.ds(off, GROUP)], idx_d.at[slot],
                         sem_i.at[slot])

    def _drain_idx(slot):
        pltpu.make_async_copy(src_hbm.at[pl.ds(0, GROUP)], idx_s.at[slot],
                              sem_i.at[slot]).wait()
        pltpu.make_async_copy(src_hbm.at[pl.ds(0, GROUP)], idx_d.at[slot],
                              sem_i.at[slot]).wait()

    def _drain_rows(rslot, sem):
        pltpu.make_async_copy(state_hbm.at[pl.ds(0, GROUP)], rows.at[rslot],
                              sem.at[rslot]).wait()

    def _fire_gather(islot, rslot):
        pltpu.async_copy(state_hbm.at[idx_s.at[islot]], rows.at[rslot],
                         sem_g.at[rslot])

    _fire_idx(0, 0)
    _fire_idx(1, 1)
    _fire_idx(2, 2)
    _drain_idx(0)
    _fire_gather(0, 0)

    def _iter(i, _):
        for j in range(4):
            g = i * 4 + j
            rslot = j % 2
            nslot = (j + 1) % 2

            @pl.when(g >= 1)
            def _():
                _drain_rows(nslot, sem_s)

            @pl.when(g + 3 < GROUPS)
            def _():
                _fire_idx(g + 3, (j + 3) % 4)

            @pl.when(g + 1 < GROUPS)
            def _():
                _drain_idx((j + 1) % 4)
                _fire_gather((j + 1) % 4, nslot)

            _drain_rows(rslot, sem_g)
            pltpu.async_copy(rows.at[rslot], shared.at[idx_d.at[j]],
                             sem_s.at[rslot], add=True)
        return 0

    lax.fori_loop(0, GROUPS // 4, _iter, 0)
    _drain_rows((GROUPS - 1) % 2, sem_s)
    plsc.subcore_barrier()

    @pl.when(cid == 0)
    def _():
        pltpu.sync_copy(shared.at[pl.ds(row0, ROWS_PER_TILE)],
                        out0.at[pl.ds(row0, ROWS_PER_TILE)])

    @pl.when(cid == 1)
    def _():
        pltpu.sync_copy(shared.at[pl.ds(row0, ROWS_PER_TILE)],
                        out1.at[pl.ds(row0, ROWS_PER_TILE)])


@jax.jit
def _sc_scatter(src, dst, state_pad):
    mesh = plsc.VectorSubcoreMesh(core_axis_name="c", subcore_axis_name="s")
    f = pl.kernel(
        _sc_body,
        out_type=[jax.ShapeDtypeStruct((N_PAD, FEAT), jnp.float32),
                  jax.ShapeDtypeStruct((N_PAD, FEAT), jnp.float32)],
        mesh=mesh,
        scratch_types=[
            pltpu.VMEM((4, GROUP), jnp.int32),
            pltpu.VMEM((4, GROUP), jnp.int32),
            pltpu.VMEM((2, GROUP, FEAT), jnp.float32),
            pltpu.VMEM((ZROWS, FEAT), jnp.float32),
            pltpu.SemaphoreType.DMA((4,)),
            pltpu.SemaphoreType.DMA((2,)),
            pltpu.SemaphoreType.DMA((2,)),
            pltpu.VMEM_SHARED((N_PAD, FEAT), jnp.float32),
        ],
    )
    return f(src, dst, state_pad)


def _tc_body(leak_ref, x_ref, s_ref, p0_ref, p1_ref, win_ref, wrec_ref, o_ref):
    aggr = p0_ref[...] + p1_ref[...]
    dn = (((1,), (1,)), ((), ()))
    pre = lax.dot_general(x_ref[...], win_ref[...], dn,
                          preferred_element_type=jnp.float32)
    pre = pre + lax.dot_general(aggr, wrec_ref[...], dn,
                                preferred_element_type=jnp.float32)
    lam = leak_ref[0, 0]
    o_ref[...] = lam * jnp.tanh(pre) + (1.0 - lam) * s_ref[...]


@jax.jit
def _tc_dense(leak, x, s, p0, p1, W_in, W_rec):
    blk = 1000
    grid = (N_NODES // blk,)
    row_spec = pl.BlockSpec((blk, FEAT), lambda i: (i, 0))
    w_spec = pl.BlockSpec((FEAT, FEAT), lambda i: (0, 0))
    return pl.pallas_call(
        _tc_body,
        grid=grid,
        in_specs=[
            pl.BlockSpec(memory_space=pltpu.SMEM),
            row_spec, row_spec, row_spec, row_spec, w_spec, w_spec,
        ],
        out_specs=row_spec,
        out_shape=jax.ShapeDtypeStruct((N_NODES, FEAT), jnp.float32),
    )(leak, x, s, p0, p1, W_in, W_rec)


def kernel(edge_index, input, state, W_in, W_rec, leakage):
    npad = E_PAD - N_EDGES
    src = jnp.concatenate([edge_index[0].astype(jnp.int32),
                           jnp.full((npad,), N_NODES, jnp.int32)])
    dst = jnp.concatenate([edge_index[1].astype(jnp.int32),
                           N_NODES + jnp.arange(npad, dtype=jnp.int32)
                           % (N_PAD - N_NODES)])
    state_pad = jnp.concatenate(
        [state, jnp.zeros((N_PAD - N_NODES, FEAT), jnp.float32)])
    p0, p1 = _sc_scatter(src, dst, state_pad)
    leak2d = jnp.asarray(leakage, jnp.float32).reshape(1, 1)
    return _tc_dense(leak2d, input, state, p0, p1, W_in, W_rec)


# speedup vs baseline: 1.0010x; 1.0010x over previous
import jax
import jax.numpy as jnp
from jax import lax
from jax.experimental import pallas as pl
from jax.experimental.pallas import tpu as pltpu
from jax.experimental.pallas import tpu_sc as plsc

N_NODES = 10000
N_EDGES = 320000
FEAT = 128
NUM_CORES = 2
NUM_SUBCORES = 16
NUM_TILES = NUM_CORES * NUM_SUBCORES
GROUP = 80
N_PAD = 10240
EDGES_PER_TILE = N_PAD
E_PAD = NUM_TILES * EDGES_PER_TILE
GROUPS = EDGES_PER_TILE // GROUP
ROWS_PER_TILE = N_PAD // NUM_SUBCORES
ZROWS = 16


def _sc_body(src_hbm, dst_hbm, state_hbm, out0, out1,
             idx_s, idx_d, rows, zbuf, sem_i, sem_g, shared):
    cid = lax.axis_index("c")
    sid = lax.axis_index("s")
    wid = cid * NUM_SUBCORES + sid

    zeros16 = jnp.zeros((16,), jnp.float32)

    def _zrow(r, _):
        def _zcol(j, _):
            zbuf[r, pl.ds(j * 16, 16)] = zeros16
            return 0
        return lax.fori_loop(0, FEAT // 16, _zcol, 0)

    lax.fori_loop(0, ZROWS, _zrow, 0)

    row0 = sid * ROWS_PER_TILE
    for b in range(ROWS_PER_TILE // ZROWS):
        pltpu.sync_copy(zbuf, shared.at[pl.ds(row0 + b * ZROWS, ZROWS)])
    plsc.subcore_barrier()

    ebase = wid * EDGES_PER_TILE

    def _fire_idx(g, slot):
        off = ebase + g * GROUP
        pltpu.async_copy(src_hbm.at[pl.ds(off, GROUP)], idx_s.at[slot],
                         sem_i.at[slot])
        pltpu.async_copy(dst_hbm.at[pl.ds(off, GROUP)], idx_d.at[slot],
                         sem_i.at[slot])

    def _drain_idx(slot):
        pltpu.make_async_copy(src_hbm.at[pl.ds(0, GROUP)], idx_s.at[slot],
                              sem_i.at[slot]).wait()
        pltpu.make_async_copy(src_hbm.at[pl.ds(0, GROUP)], idx_d.at[slot],
                              sem_i.at[slot]).wait()

    def _drain_rows(rslot, sem):
        pltpu.make_async_copy(state_hbm.at[pl.ds(0, GROUP)], rows.at[rslot],
                              sem.at[rslot]).wait()

    def _fire_gather(islot, rslot):
        pltpu.async_copy(state_hbm.at[idx_s.at[islot]], rows.at[rslot],
                         sem_g.at[rslot])

    _fire_idx(0, 0)
    _fire_idx(1, 1)
    _fire_idx(2, 2)
    _drain_idx(0)
    _fire_gather(0, 0)

    def _iter(i, _):
        for j in range(4):
            g = i * 4 + j
            rslot = j % 2
            nslot = (j + 1) % 2

            @pl.when(g + 3 < GROUPS)
            def _():
                _fire_idx(g + 3, (j + 3) % 4)

            @pl.when(g + 1 < GROUPS)
            def _():
                _drain_idx((j + 1) % 4)
                _fire_gather((j + 1) % 4, nslot)

            _drain_rows(rslot, sem_g)
            pltpu.sync_copy(rows.at[rslot], shared.at[idx_d.at[j]], add=True)
        return 0

    lax.fori_loop(0, GROUPS // 4, _iter, 0)
    plsc.subcore_barrier()

    @pl.when(cid == 0)
    def _():
        pltpu.sync_copy(shared.at[pl.ds(row0, ROWS_PER_TILE)],
                        out0.at[pl.ds(row0, ROWS_PER_TILE)])

    @pl.when(cid == 1)
    def _():
        pltpu.sync_copy(shared.at[pl.ds(row0, ROWS_PER_TILE)],
                        out1.at[pl.ds(row0, ROWS_PER_TILE)])


@jax.jit
def _sc_scatter(src, dst, state_pad):
    mesh = plsc.VectorSubcoreMesh(core_axis_name="c", subcore_axis_name="s")
    f = pl.kernel(
        _sc_body,
        out_type=[jax.ShapeDtypeStruct((N_PAD, FEAT), jnp.float32),
                  jax.ShapeDtypeStruct((N_PAD, FEAT), jnp.float32)],
        mesh=mesh,
        scratch_types=[
            pltpu.VMEM((4, GROUP), jnp.int32),
            pltpu.VMEM((4, GROUP), jnp.int32),
            pltpu.VMEM((2, GROUP, FEAT), jnp.float32),
            pltpu.VMEM((ZROWS, FEAT), jnp.float32),
            pltpu.SemaphoreType.DMA((4,)),
            pltpu.SemaphoreType.DMA((2,)),
            pltpu.VMEM_SHARED((N_PAD, FEAT), jnp.float32),
        ],
    )
    return f(src, dst, state_pad)


def _tc_body(leak_ref, x_ref, s_ref, p0_ref, p1_ref, win_ref, wrec_ref, o_ref):
    aggr = p0_ref[...] + p1_ref[...]
    dn = (((1,), (1,)), ((), ()))
    pre = lax.dot_general(x_ref[...], win_ref[...], dn,
                          preferred_element_type=jnp.float32)
    pre = pre + lax.dot_general(aggr, wrec_ref[...], dn,
                                preferred_element_type=jnp.float32)
    lam = leak_ref[0, 0]
    o_ref[...] = lam * jnp.tanh(pre) + (1.0 - lam) * s_ref[...]


@jax.jit
def _tc_dense(leak, x, s, p0, p1, W_in, W_rec):
    blk = 1000
    grid = (N_NODES // blk,)
    row_spec = pl.BlockSpec((blk, FEAT), lambda i: (i, 0))
    w_spec = pl.BlockSpec((FEAT, FEAT), lambda i: (0, 0))
    return pl.pallas_call(
        _tc_body,
        grid=grid,
        in_specs=[
            pl.BlockSpec(memory_space=pltpu.SMEM),
            row_spec, row_spec, row_spec, row_spec, w_spec, w_spec,
        ],
        out_specs=row_spec,
        out_shape=jax.ShapeDtypeStruct((N_NODES, FEAT), jnp.float32),
    )(leak, x, s, p0, p1, W_in, W_rec)


def kernel(edge_index, input, state, W_in, W_rec, leakage):
    npad = E_PAD - N_EDGES
    src = jnp.concatenate([edge_index[0].astype(jnp.int32),
                           jnp.full((npad,), N_NODES, jnp.int32)])
    dst = jnp.concatenate([edge_index[1].astype(jnp.int32),
                           N_NODES + jnp.arange(npad, dtype=jnp.int32)
                           % (N_PAD - N_NODES)])
    state_pad = jnp.concatenate(
        [state, jnp.zeros((N_PAD - N_NODES, FEAT), jnp.float32)])
    p0, p1 = _sc_scatter(src, dst, state_pad)
    leak2d = jnp.asarray(leakage, jnp.float32).reshape(1, 1)
    return _tc_dense(leak2d, input, state, p0, p1, W_in, W_rec)


# speedup vs baseline: 2.7094x; 2.7067x over previous
import jax
import jax.numpy as jnp
from jax import lax
from jax.experimental import pallas as pl
from jax.experimental.pallas import tpu as pltpu
from jax.experimental.pallas import tpu_sc as plsc

N_NODES = 10000
N_EDGES = 320000
FEAT = 128
NUM_CORES = 2
NUM_SUBCORES = 16
NUM_TILES = NUM_CORES * NUM_SUBCORES
GROUP = 80
N_PAD = 10240
EDGES_PER_TILE = N_EDGES // NUM_TILES
GROUPS = EDGES_PER_TILE // GROUP
ROWS_PER_TILE = N_PAD // NUM_SUBCORES


def _sc_body(src_hbm, dst_hbm, state_hbm, out0, out1,
             idx_s, idx_d, rows, sem_i, sem_g, shared):
    cid = lax.axis_index("c")
    sid = lax.axis_index("s")
    wid = cid * NUM_SUBCORES + sid

    zeros16 = jnp.zeros((16,), jnp.float32)

    def _zrow(r, _):
        def _zcol(j, _):
            rows[0, r, pl.ds(j * 16, 16)] = zeros16
            return 0
        return lax.fori_loop(0, FEAT // 16, _zcol, 0)

    lax.fori_loop(0, GROUP, _zrow, 0)

    row0 = sid * ROWS_PER_TILE
    for b in range(ROWS_PER_TILE // GROUP):
        pltpu.sync_copy(rows.at[0], shared.at[pl.ds(row0 + b * GROUP, GROUP)])
    plsc.subcore_barrier()

    ebase = wid * EDGES_PER_TILE

    def _fire_idx(g, slot):
        off = ebase + g * GROUP
        pltpu.async_copy(src_hbm.at[pl.ds(off, GROUP)], idx_s.at[slot],
                         sem_i.at[slot])
        pltpu.async_copy(dst_hbm.at[pl.ds(off, GROUP)], idx_d.at[slot],
                         sem_i.at[slot])

    def _drain_idx(slot):
        pltpu.make_async_copy(src_hbm.at[pl.ds(0, GROUP)], idx_s.at[slot],
                              sem_i.at[slot]).wait()
        pltpu.make_async_copy(src_hbm.at[pl.ds(0, GROUP)], idx_d.at[slot],
                              sem_i.at[slot]).wait()

    def _drain_rows(rslot, sem):
        pltpu.make_async_copy(state_hbm.at[pl.ds(0, GROUP)], rows.at[rslot],
                              sem.at[rslot]).wait()

    def _fire_gather(islot, rslot):
        pltpu.async_copy(state_hbm.at[idx_s.at[islot]], rows.at[rslot],
                         sem_g.at[rslot])

    _fire_idx(0, 0)
    _fire_idx(1, 1)
    _fire_idx(2, 2)
    _drain_idx(0)
    _fire_gather(0, 0)

    def _iter(i, _):
        for j in range(4):
            g = i * 4 + j
            rslot = j % 2
            nslot = (j + 1) % 2

            @pl.when(g + 3 < GROUPS)
            def _():
                _fire_idx(g + 3, (j + 3) % 4)

            @pl.when(g + 1 < GROUPS)
            def _():
                _drain_idx((j + 1) % 4)
                _fire_gather((j + 1) % 4, nslot)

            _drain_rows(rslot, sem_g)
            pltpu.sync_copy(rows.at[rslot], shared.at[idx_d.at[j]], add=True)
        return 0

    lax.fori_loop(0, GROUPS // 4, _iter, 0)
    _drain_rows(0, sem_g)
    pltpu.sync_copy(rows.at[0], shared.at[idx_d.at[0]], add=True)
    plsc.subcore_barrier()

    @pl.when(cid == 0)
    def _():
        pltpu.sync_copy(shared.at[pl.ds(row0, ROWS_PER_TILE)],
                        out0.at[pl.ds(row0, ROWS_PER_TILE)])

    @pl.when(cid == 1)
    def _():
        pltpu.sync_copy(shared.at[pl.ds(row0, ROWS_PER_TILE)],
                        out1.at[pl.ds(row0, ROWS_PER_TILE)])


@jax.jit
def _sc_scatter(src, dst, state):
    mesh = plsc.VectorSubcoreMesh(core_axis_name="c", subcore_axis_name="s")
    f = pl.kernel(
        _sc_body,
        out_type=[jax.ShapeDtypeStruct((N_PAD, FEAT), jnp.float32),
                  jax.ShapeDtypeStruct((N_PAD, FEAT), jnp.float32)],
        mesh=mesh,
        scratch_types=[
            pltpu.VMEM((4, GROUP), jnp.int32),
            pltpu.VMEM((4, GROUP), jnp.int32),
            pltpu.VMEM((2, GROUP, FEAT), jnp.float32),
            pltpu.SemaphoreType.DMA((4,)),
            pltpu.SemaphoreType.DMA((2,)),
            pltpu.VMEM_SHARED((N_PAD, FEAT), jnp.float32),
        ],
    )
    return f(src, dst, state)


def _tc_body(leak_ref, x_ref, s_ref, p0_ref, p1_ref, win_ref, wrec_ref, o_ref):
    aggr = p0_ref[...] + p1_ref[...]
    dn = (((1,), (1,)), ((), ()))
    pre = lax.dot_general(x_ref[...], win_ref[...], dn,
                          preferred_element_type=jnp.float32)
    pre = pre + lax.dot_general(aggr, wrec_ref[...], dn,
                                preferred_element_type=jnp.float32)
    lam = leak_ref[0, 0]
    o_ref[...] = lam * jnp.tanh(pre) + (1.0 - lam) * s_ref[...]


@jax.jit
def _tc_dense(leak, x, s, p0, p1, W_in, W_rec):
    blk = 1000
    grid = (N_NODES // blk,)
    row_spec = pl.BlockSpec((blk, FEAT), lambda i: (i, 0))
    w_spec = pl.BlockSpec((FEAT, FEAT), lambda i: (0, 0))
    return pl.pallas_call(
        _tc_body,
        grid=grid,
        in_specs=[
            pl.BlockSpec(memory_space=pltpu.SMEM),
            row_spec, row_spec, row_spec, row_spec, w_spec, w_spec,
        ],
        out_specs=row_spec,
        out_shape=jax.ShapeDtypeStruct((N_NODES, FEAT), jnp.float32),
    )(leak, x, s, p0, p1, W_in, W_rec)


def kernel(edge_index, input, state, W_in, W_rec, leakage):
    src = edge_index[0].astype(jnp.int32)
    dst = edge_index[1].astype(jnp.int32)
    p0, p1 = _sc_scatter(src, dst, state)
    leak2d = jnp.asarray(leakage, jnp.float32).reshape(1, 1)
    return _tc_dense(leak2d, input, state, p0, p1, W_in, W_rec)
